# baseline (speedup 1.0000x reference)
import functools

import jax
import jax.numpy as jnp
from jax import lax
from jax.experimental import pallas as pl
from jax.experimental.pallas import tpu as pltpu
from jax.experimental.pallas import tpu_sc as plsc

B = 1024
N = 256
S = 16
H = 128
K = 16
L = 16
NC = 2
NS = 16
NW = NC * NS
BPW = B // NW
NCH = H // L
NQ = 4
RPQ = BPW * K // NQ
SPQ = BPW // NQ
PAR = 4 * H
WKN = PAR + L


def _lrelu(x):
    return jnp.where(x >= 0, x, 0.2 * x)


def _splat(ref, s):
    return plsc.load_gather(ref, [jnp.full((L,), s, jnp.int32)])


@functools.partial(
    pl.kernel,
    out_type=jax.ShapeDtypeStruct((B * K,), jnp.float32),
    mesh=plsc.VectorSubcoreMesh(core_axis_name="c", subcore_axis_name="s"),
    compiler_params=pltpu.CompilerParams(needs_layout_passes=False),
    scratch_types=[
        pltpu.VMEM((BPW,), jnp.int32),
        pltpu.VMEM((NQ, RPQ), jnp.int32),
        pltpu.VMEM((BPW,), jnp.int32),
        pltpu.VMEM((BPW * K, H), jnp.float32),
        pltpu.VMEM((BPW, H), jnp.float32),
        pltpu.VMEM((BPW, H), jnp.float32),
        pltpu.VMEM((H, H), jnp.float32),
        pltpu.VMEM((WKN,), jnp.float32),
        pltpu.VMEM((H,), jnp.float32),
        pltpu.VMEM((BPW,), jnp.float32),
        pltpu.VMEM((K * L,), jnp.float32),
        pltpu.VMEM((BPW * K,), jnp.float32),
        pltpu.SemaphoreType.DMA,
        pltpu.SemaphoreType.DMA,
        pltpu.SemaphoreType.DMA,
        pltpu.SemaphoreType.DMA,
        pltpu.SemaphoreType.DMA,
    ],
)
def _sc_fwd(nodes_hbm, subs_hbm, obs_hbm, subc_hbm, wp_hbm, w_hbm, out_hbm,
            idx_v, nidx_v, subc_v, nodes_v, subs_v, obs_v, wp_v, wk_v, vv_v,
            hs_v, hmat_v, out_v, sem_s, sq0, sq1, sq2, sq3):
    wid = lax.axis_index("s") * NC + lax.axis_index("c")
    base = wid * BPW
    iota = lax.iota(jnp.int32, L)
    iota16 = iota * L

    pltpu.sync_copy(subc_hbm.at[pl.ds(base, BPW)], subc_v)
    subh = [subc_v[pl.ds(0, L)], subc_v[pl.ds(L, L)]]
    for half in range(BPW // L):
        idx_v[pl.ds(half * L, L)] = (base + half * L + iota) * S + subh[half]
    for j in range(BPW):
        sub_j = subh[j // L][j % L]
        nbase = (base + j) * N + sub_j * K
        q, r = divmod(j * K, RPQ)
        nidx_v[q, pl.ds(r, K)] = nbase + iota

    sems = [sq0, sq1, sq2, sq3]
    cps = [
        pltpu.async_copy(
            nodes_hbm.at[nidx_v.at[q]],
            nodes_v.at[pl.ds(q * RPQ, RPQ), :],
            sems[q],
        )
        for q in range(NQ)
    ]
    cp_s = pltpu.async_copy(subs_hbm.at[idx_v], subs_v, sem_s)
    pltpu.sync_copy(obs_hbm.at[pl.ds(base, BPW)], obs_v)
    pltpu.sync_copy(wp_hbm, wp_v)
    pltpu.sync_copy(w_hbm, wk_v)

    w1c = [wk_v[pl.ds(c * L, L)] for c in range(NCH)]

    def fold_body(g, carry):
        for r in range(L):
            acc = wp_v[g * L + r, pl.ds(0, L)] * w1c[0]
            for c in range(1, NCH):
                acc = acc + wp_v[g * L + r, pl.ds(c * L, L)] * w1c[c]
            hmat_v[pl.ds(r * L, L)] = acc
        vchunk = plsc.load_gather(hmat_v, [iota16])
        for c in range(1, L):
            vchunk = vchunk + plsc.load_gather(hmat_v, [iota16 + c])
        vv_v[pl.ds(pl.multiple_of(g * L, L), L)] = vchunk
        return carry

    lax.fori_loop(0, H // L, fold_body, 0)

    accb = wk_v[pl.ds(PAR - H, L)] * w1c[0]
    for c in range(1, NCH):
        accb = accb + wk_v[pl.ds(PAR - H + c * L, L)] * w1c[c]
    c0 = jnp.sum(accb)
    parv = wk_v[pl.ds(PAR, L)]
    a_src = parv[0]
    a_dst = parv[1]
    b_gat = parv[2]

    cp_s.wait()
    w2c = [wk_v[pl.ds(H + c * L, L)] for c in range(NCH)]
    for grp in range(BPW // L):
        for t in range(L):
            s = grp * L + t
            acc = (obs_v[s, pl.ds(0, L)] * vv_v[pl.ds(0, L)]
                   + subs_v[s, pl.ds(0, L)] * w2c[0])
            for c in range(1, NCH):
                acc = acc + obs_v[s, pl.ds(c * L, L)] * vv_v[pl.ds(c * L, L)]
                acc = acc + subs_v[s, pl.ds(c * L, L)] * w2c[c]
            hmat_v[pl.ds(t * L, L)] = acc
        hsg = plsc.load_gather(hmat_v, [iota16])
        for c in range(1, L):
            hsg = hsg + plsc.load_gather(hmat_v, [iota16 + c])
        hs_v[pl.ds(grp * L, L)] = hsg + c0

    w3c = [wk_v[pl.ds(2 * H + c * L, L)] for c in range(NCH)]

    def mk_body():
        def body(s, carry):
            srow = s * K
            for k in range(K):
                a2 = nodes_v[srow + k, pl.ds(0, L)] * w3c[0]
                for c in range(1, NCH):
                    a2 = a2 + nodes_v[srow + k, pl.ds(c * L, L)] * w3c[c]
                hmat_v[pl.ds(k * L, L)] = a2
            hvec = plsc.load_gather(hmat_v, [iota16])
            for c in range(1, L):
                hvec = hvec + plsc.load_gather(hmat_v, [iota16 + c])
            hvec = hvec + _splat(hs_v, s)
            svec = hvec * a_src
            dvec = hvec * a_dst
            mvec = _lrelu(jnp.max(svec) + dvec)
            den = None
            num = None
            for i in range(K):
                e = _lrelu(svec[i] + dvec)
                w = jnp.exp(e - mvec)
                den = w if den is None else den + w
                num = w * hvec[i] if num is None else num + w * hvec[i]
            out_v[pl.ds(pl.multiple_of(srow, L), L)] = num / den + b_gat
            return carry
        return body

    body = mk_body()
    for q in range(NQ):
        cps[q].wait()
        lax.fori_loop(q * SPQ, (q + 1) * SPQ, body, 0)

    pltpu.sync_copy(out_v, out_hbm.at[pl.ds(base * K, BPW * K)])


def kernel(org_obs, node_embeddings, substation_embeddings, sub_choice,
           W_proj, b_proj, W_gat, a_src, a_dst, b_gat):
    wpacked = jnp.concatenate([
        W_gat[:, 0], b_proj,
        a_src.astype(jnp.float32), a_dst.astype(jnp.float32),
        b_gat.astype(jnp.float32),
        jnp.zeros((WKN - PAR - 3,), jnp.float32),
    ])
    nodes_flat = node_embeddings.reshape(B * N, H)
    subs_flat = substation_embeddings.reshape(B * S, H)
    subc = sub_choice.reshape(B).astype(jnp.int32)

    out = _sc_fwd(nodes_flat, subs_flat, org_obs, subc, W_proj, wpacked)
    return (out.reshape(B * K, 1), sub_choice)


# speedup vs baseline: 1.3785x; 1.3785x over previous
import functools

import jax
import jax.numpy as jnp
from jax import lax
from jax.experimental import pallas as pl
from jax.experimental.pallas import tpu as pltpu
from jax.experimental.pallas import tpu_sc as plsc

B = 1024
N = 256
S = 16
H = 128
K = 16
L = 16
NC = 2
NS = 16
NW = NC * NS
BPW = B // NW
NCH = H // L
NQ = 4
RPQ = BPW * K // NQ
SPQ = BPW // NQ
PAR = 3 * H
WKN = PAR + L


def _lrelu(x):
    return jnp.where(x >= 0, x, 0.2 * x)


@functools.partial(
    pl.kernel,
    out_type=jax.ShapeDtypeStruct((B * K,), jnp.float32),
    mesh=plsc.VectorSubcoreMesh(core_axis_name="c", subcore_axis_name="s"),
    compiler_params=pltpu.CompilerParams(needs_layout_passes=False),
    scratch_types=[
        pltpu.VMEM((BPW,), jnp.int32),
        pltpu.VMEM((NQ, RPQ), jnp.int32),
        pltpu.VMEM((BPW,), jnp.int32),
        pltpu.VMEM((BPW * K, H), jnp.float32),
        pltpu.VMEM((BPW, H), jnp.float32),
        pltpu.VMEM((BPW, H), jnp.float32),
        pltpu.VMEM((WKN,), jnp.float32),
        pltpu.VMEM((BPW * K,), jnp.float32),
        pltpu.SemaphoreType.DMA,
        pltpu.SemaphoreType.DMA,
        pltpu.SemaphoreType.DMA,
        pltpu.SemaphoreType.DMA,
        pltpu.SemaphoreType.DMA,
    ],
)
def _sc_fwd(nodes_hbm, subs_hbm, obs_hbm, subc_hbm, w_hbm, out_hbm,
            idx_v, nidx_v, subc_v, nodes_v, subs_v, obs_v, wk_v,
            out_v, sem_s, sq0, sq1, sq2, sq3):
    wid = lax.axis_index("s") * NC + lax.axis_index("c")
    base = wid * BPW
    iota = lax.iota(jnp.int32, L)

    pltpu.sync_copy(subc_hbm.at[pl.ds(base, BPW)], subc_v)
    subh = [subc_v[pl.ds(0, L)], subc_v[pl.ds(L, L)]]
    for half in range(BPW // L):
        idx_v[pl.ds(half * L, L)] = (base + half * L + iota) * S + subh[half]
    for j in range(BPW):
        sub_j = subh[j // L][j % L]
        nbase = (base + j) * N + sub_j * K
        q, r = divmod(j * K, RPQ)
        nidx_v[q, pl.ds(r, K)] = nbase + iota

    sems = [sq0, sq1, sq2, sq3]
    cps = [
        pltpu.async_copy(
            nodes_hbm.at[nidx_v.at[q]],
            nodes_v.at[pl.ds(q * RPQ, RPQ), :],
            sems[q],
        )
        for q in range(NQ)
    ]
    cp_s = pltpu.async_copy(subs_hbm.at[idx_v], subs_v, sem_s)
    pltpu.sync_copy(obs_hbm.at[pl.ds(base, BPW)], obs_v)
    pltpu.sync_copy(w_hbm, wk_v)

    par = wk_v[pl.ds(PAR, L)]
    c0 = par[0]
    a_src = par[1]
    a_dst = par[2]
    b_gat = par[3]
    lane = iota

    def body(s, carry):
        acc = (obs_v[s, pl.ds(0, L)] * wk_v[pl.ds(0, L)]
               + subs_v[s, pl.ds(0, L)] * wk_v[pl.ds(H, L)])
        for c in range(1, NCH):
            acc = acc + obs_v[s, pl.ds(c * L, L)] * wk_v[pl.ds(c * L, L)]
            acc = acc + subs_v[s, pl.ds(c * L, L)] * wk_v[pl.ds(H + c * L, L)]
        hs = jnp.sum(acc) + c0

        srow = s * K
        hvec = None
        for k in range(K):
            a2 = nodes_v[srow + k, pl.ds(0, L)] * wk_v[pl.ds(2 * H, L)]
            for c in range(1, NCH):
                a2 = a2 + (nodes_v[srow + k, pl.ds(c * L, L)]
                           * wk_v[pl.ds(2 * H + c * L, L)])
            hk = jnp.full((L,), jnp.sum(a2) + hs)
            hvec = hk if hvec is None else jnp.where(lane == k, hk, hvec)

        svec = hvec * a_src
        dvec = hvec * a_dst
        mvec = _lrelu(jnp.max(svec) + dvec)

        den = None
        num = None
        for i in range(K):
            e = _lrelu(svec[i] + dvec)
            w = jnp.exp(e - mvec)
            den = w if den is None else den + w
            num = w * hvec[i] if num is None else num + w * hvec[i]
        out_v[pl.ds(pl.multiple_of(srow, L), L)] = num / den + b_gat
        return carry

    cp_s.wait()
    for q in range(NQ):
        cps[q].wait()
        lax.fori_loop(q * SPQ, (q + 1) * SPQ, body, 0)

    pltpu.sync_copy(out_v, out_hbm.at[pl.ds(base * K, BPW * K)])


def kernel(org_obs, node_embeddings, substation_embeddings, sub_choice,
           W_proj, b_proj, W_gat, a_src, a_dst, b_gat):
    w1 = W_gat[:H, 0]
    v = W_proj @ w1
    c0 = jnp.dot(b_proj, w1)
    wpacked = jnp.concatenate([
        v, W_gat[H:, 0],
        c0[None], a_src.astype(jnp.float32), a_dst.astype(jnp.float32),
        b_gat.astype(jnp.float32),
        jnp.zeros((L - 4,), jnp.float32),
    ])
    nodes_flat = node_embeddings.reshape(B * N, H)
    subs_flat = substation_embeddings.reshape(B * S, H)
    subc = sub_choice.reshape(B).astype(jnp.int32)

    out = _sc_fwd(nodes_flat, subs_flat, org_obs, subc, wpacked)
    return (out.reshape(B * K, 1), sub_choice)


# speedup vs baseline: 1.3915x; 1.0095x over previous
import functools

import jax
import jax.numpy as jnp
from jax import lax
from jax.experimental import pallas as pl
from jax.experimental.pallas import tpu as pltpu
from jax.experimental.pallas import tpu_sc as plsc

B = 1024
N = 256
S = 16
H = 128
K = 16
L = 16
NC = 2
NS = 16
NW = NC * NS
BPW = B // NW
NCH = H // L
NQ = 4
RPQ = BPW * K // NQ
SPQ = BPW // NQ
PAR = 3 * H
WKN = PAR + L


def _lrelu(x):
    return jnp.where(x >= 0, x, 0.2 * x)


@functools.partial(
    pl.kernel,
    out_type=jax.ShapeDtypeStruct((B * K,), jnp.float32),
    mesh=plsc.VectorSubcoreMesh(core_axis_name="c", subcore_axis_name="s"),
    compiler_params=pltpu.CompilerParams(needs_layout_passes=False),
    scratch_types=[
        pltpu.VMEM((BPW,), jnp.int32),
        pltpu.VMEM((NQ, RPQ), jnp.int32),
        pltpu.VMEM((BPW,), jnp.int32),
        pltpu.VMEM((BPW * K, H), jnp.float32),
        pltpu.VMEM((BPW, H), jnp.float32),
        pltpu.VMEM((BPW, H), jnp.float32),
        pltpu.VMEM((WKN,), jnp.float32),
        pltpu.VMEM((BPW * K,), jnp.float32),
        pltpu.SemaphoreType.DMA,
        pltpu.SemaphoreType.DMA,
        pltpu.SemaphoreType.DMA,
        pltpu.SemaphoreType.DMA,
        pltpu.SemaphoreType.DMA,
    ],
)
def _sc_fwd(nodes_hbm, subs_hbm, obs_hbm, subc_hbm, w_hbm, out_hbm,
            idx_v, nidx_v, subc_v, nodes_v, subs_v, obs_v, wk_v,
            out_v, sem_s, sq0, sq1, sq2, sq3):
    wid = lax.axis_index("s") * NC + lax.axis_index("c")
    base = wid * BPW
    iota = lax.iota(jnp.int32, L)

    pltpu.sync_copy(subc_hbm.at[pl.ds(base, BPW)], subc_v)
    subh = [subc_v[pl.ds(0, L)], subc_v[pl.ds(L, L)]]
    for half in range(BPW // L):
        idx_v[pl.ds(half * L, L)] = (base + half * L + iota) * S + subh[half]
    for j in range(BPW):
        sub_j = subh[j // L][j % L]
        nbase = (base + j) * N + sub_j * K
        q, r = divmod(j * K, RPQ)
        nidx_v[q, pl.ds(r, K)] = nbase + iota

    sems = [sq0, sq1, sq2, sq3]
    cps = [
        pltpu.async_copy(
            nodes_hbm.at[nidx_v.at[q]],
            nodes_v.at[pl.ds(q * RPQ, RPQ), :],
            sems[q],
        )
        for q in range(NQ)
    ]
    cp_s = pltpu.async_copy(subs_hbm.at[idx_v], subs_v, sem_s)
    pltpu.sync_copy(obs_hbm.at[pl.ds(base, BPW)], obs_v)
    pltpu.sync_copy(w_hbm, wk_v)

    par = wk_v[pl.ds(PAR, L)]
    c0 = par[0]
    a_src = par[1]
    a_dst = par[2]
    b_gat = par[3]
    lane = iota

    def sample_out(s):
        acc = (obs_v[s, pl.ds(0, L)] * wk_v[pl.ds(0, L)]
               + subs_v[s, pl.ds(0, L)] * wk_v[pl.ds(H, L)])
        for c in range(1, NCH):
            acc = acc + obs_v[s, pl.ds(c * L, L)] * wk_v[pl.ds(c * L, L)]
            acc = acc + subs_v[s, pl.ds(c * L, L)] * wk_v[pl.ds(H + c * L, L)]
        hs = jnp.sum(acc) + c0

        srow = s * K
        hvec = None
        for k in range(K):
            a2 = nodes_v[srow + k, pl.ds(0, L)] * wk_v[pl.ds(2 * H, L)]
            for c in range(1, NCH):
                a2 = a2 + (nodes_v[srow + k, pl.ds(c * L, L)]
                           * wk_v[pl.ds(2 * H + c * L, L)])
            hk = jnp.full((L,), jnp.sum(a2) + hs)
            hvec = hk if hvec is None else jnp.where(lane == k, hk, hvec)

        svec = hvec * a_src
        dvec = hvec * a_dst
        mvec = _lrelu(jnp.max(svec) + dvec)

        den = None
        num = None
        for i in range(K):
            e = _lrelu(svec[i] + dvec)
            w = jnp.exp(e - mvec)
            den = w if den is None else den + w
            num = w * hvec[i] if num is None else num + w * hvec[i]
        return num / den + b_gat

    def body(s, carry):
        r0 = sample_out(s)
        r1 = sample_out(s + SPQ // 2)
        out_v[pl.ds(pl.multiple_of(s * K, L), L)] = r0
        out_v[pl.ds(pl.multiple_of((s + SPQ // 2) * K, L), L)] = r1
        return carry

    cp_s.wait()
    for q in range(NQ):
        cps[q].wait()
        lax.fori_loop(q * SPQ, q * SPQ + SPQ // 2, body, 0)

    pltpu.sync_copy(out_v, out_hbm.at[pl.ds(base * K, BPW * K)])


def kernel(org_obs, node_embeddings, substation_embeddings, sub_choice,
           W_proj, b_proj, W_gat, a_src, a_dst, b_gat):
    w1 = W_gat[:H, 0]
    v = W_proj @ w1
    c0 = jnp.dot(b_proj, w1)
    wpacked = jnp.concatenate([
        v, W_gat[H:, 0],
        c0[None], a_src.astype(jnp.float32), a_dst.astype(jnp.float32),
        b_gat.astype(jnp.float32),
        jnp.zeros((L - 4,), jnp.float32),
    ])
    nodes_flat = node_embeddings.reshape(B * N, H)
    subs_flat = substation_embeddings.reshape(B * S, H)
    subc = sub_choice.reshape(B).astype(jnp.int32)

    out = _sc_fwd(nodes_flat, subs_flat, org_obs, subc, wpacked)
    return (out.reshape(B * K, 1), sub_choice)
